# baseline (speedup 1.0000x reference)
import jax
import jax.numpy as jnp
from jax.experimental import pallas as pl
from jax.experimental.pallas import tpu as pltpu

BATCH = 16
N = 512
EMB_DIM = 128
FEAT_DIM = 32
HEADS = 4


def _gat_kernel(x_ref, adj_ref, w_ref, b_ref, wsrc_ref, wdst_ref,
                wh_ref, bh_ref, attn_ref, out_ref):
  for bi in range(2):
    x = x_ref[bi]
    adj_f = adj_ref[bi]
    outs = []
    for hi in range(HEADS):
        h = jnp.dot(x, w_ref[hi], preferred_element_type=jnp.float32)
        th = jnp.tanh(h)
        s = jnp.sum(th * wsrc_ref[0, hi], axis=1, keepdims=True)
        d = jnp.sum(th * wdst_ref[0, hi], axis=1, keepdims=True)
        drow = d.T
        sm = s + d
        m = jnp.maximum(sm, 0.2 * sm)
        e1 = jnp.exp(jnp.minimum(s - m, 80.0))
        e2 = jnp.exp(jnp.minimum(0.2 * s - m, 80.0))
        f1 = jnp.exp(jnp.minimum(drow, 80.0))
        f2 = jnp.exp(jnp.minimum(0.2 * drow, 80.0))
        e = jnp.minimum(jnp.maximum(e1 * f1, e2 * f2),
                        jnp.float32(1e30)) * adj_f
        p = e * (1.0 / jnp.sum(e, axis=1, keepdims=True))
        attn_ref[bi, hi] = p
        fo = jnp.dot(p, h, preferred_element_type=jnp.float32) + b_ref[0]
        outs.append(jnp.where(fo > 0, fo, jnp.exp(jnp.minimum(fo, 0.0)) - 1.0))
    fo_cat = jnp.concatenate(outs, axis=1)
    gate = jax.nn.sigmoid(
        jnp.dot(x, wh_ref[...], preferred_element_type=jnp.float32)
        + bh_ref[0])
    out_ref[bi] = gate * fo_cat + (1.0 - gate) * x


def kernel(doc_sents_h, doc_len, adj, W, b, w_src, w_dst, Wh, bh):
    del doc_len
    b2 = b.reshape(1, FEAT_DIM)
    wsrc = w_src.reshape(1, HEADS, FEAT_DIM)
    wdst = w_dst.reshape(1, HEADS, FEAT_DIM)
    bh2 = bh.reshape(1, HEADS * FEAT_DIM)

    attn, feat_out = pl.pallas_call(
        _gat_kernel,
        grid=(BATCH // 2,),
        in_specs=[
            pl.BlockSpec((2, N, EMB_DIM), lambda bi: (bi, 0, 0)),
            pl.BlockSpec((2, N, N), lambda bi: (bi, 0, 0)),
            pl.BlockSpec((HEADS, EMB_DIM, FEAT_DIM), lambda bi: (0, 0, 0)),
            pl.BlockSpec((1, FEAT_DIM), lambda bi: (0, 0)),
            pl.BlockSpec((1, HEADS, FEAT_DIM), lambda bi: (0, 0, 0)),
            pl.BlockSpec((1, HEADS, FEAT_DIM), lambda bi: (0, 0, 0)),
            pl.BlockSpec((EMB_DIM, HEADS * FEAT_DIM), lambda bi: (0, 0)),
            pl.BlockSpec((1, HEADS * FEAT_DIM), lambda bi: (0, 0)),
        ],
        out_specs=[
            pl.BlockSpec((2, HEADS, N, N), lambda bi: (bi, 0, 0, 0)),
            pl.BlockSpec((2, N, HEADS * FEAT_DIM), lambda bi: (bi, 0, 0)),
        ],
        out_shape=[
            jax.ShapeDtypeStruct((BATCH, HEADS, N, N), jnp.float32),
            jax.ShapeDtypeStruct((BATCH, N, HEADS * FEAT_DIM), jnp.float32),
        ],
        compiler_params=pltpu.CompilerParams(
            dimension_semantics=("parallel",),
        ),
    )(doc_sents_h, adj, W, b2, wsrc, wdst, Wh, bh2)
    return feat_out, attn


# speedup vs baseline: 1.1087x; 1.1087x over previous
import jax
import jax.numpy as jnp
from jax.experimental import pallas as pl
from jax.experimental.pallas import tpu as pltpu

BATCH = 16
N = 512
EMB_DIM = 128
FEAT_DIM = 32
HEADS = 4


def _gat_kernel(x_ref, adj_ref, w_ref, b_ref, wsrc_ref, wdst_ref,
                wh_ref, bh_ref, attn_ref, out_ref):
    x = x_ref[0]
    adj_f = adj_ref[0]
    hs, ss, ds = [], [], []
    for hi in range(HEADS):
        h = jnp.dot(x, w_ref[hi], preferred_element_type=jnp.float32)
        hs.append(h)
        th = jnp.tanh(h)
        ss.append(jnp.sum(th * wsrc_ref[0, hi], axis=1, keepdims=True))
        ds.append(jnp.sum(th * wdst_ref[0, hi], axis=1, keepdims=True))
    outs = []
    for hi in range(HEADS):
        h, s, d = hs[hi], ss[hi], ds[hi]
        drow = d.T
        sm = s + d
        m = jnp.maximum(sm, 0.2 * sm)
        e1 = jnp.exp(jnp.minimum(s - m, 80.0))
        e2 = jnp.exp(jnp.minimum(0.2 * s - m, 80.0))
        f1 = jnp.exp(jnp.minimum(drow, 80.0))
        f2 = jnp.exp(jnp.minimum(0.2 * drow, 80.0))
        e = jnp.minimum(jnp.maximum(e1 * f1, e2 * f2),
                        jnp.float32(1e30)) * adj_f
        p = e * (1.0 / jnp.sum(e, axis=1, keepdims=True))
        attn_ref[0, hi] = p
        fo = jnp.dot(p, h, preferred_element_type=jnp.float32) + b_ref[0]
        outs.append(jnp.where(fo > 0, fo, jnp.exp(jnp.minimum(fo, 0.0)) - 1.0))
    fo_cat = jnp.concatenate(outs, axis=1)
    gate = jax.nn.sigmoid(
        jnp.dot(x, wh_ref[...], preferred_element_type=jnp.float32)
        + bh_ref[0])
    out_ref[0] = gate * fo_cat + (1.0 - gate) * x


def kernel(doc_sents_h, doc_len, adj, W, b, w_src, w_dst, Wh, bh):
    del doc_len
    b2 = b.reshape(1, FEAT_DIM)
    wsrc = w_src.reshape(1, HEADS, FEAT_DIM)
    wdst = w_dst.reshape(1, HEADS, FEAT_DIM)
    bh2 = bh.reshape(1, HEADS * FEAT_DIM)

    attn, feat_out = pl.pallas_call(
        _gat_kernel,
        grid=(BATCH,),
        in_specs=[
            pl.BlockSpec((1, N, EMB_DIM), lambda bi: (bi, 0, 0)),
            pl.BlockSpec((1, N, N), lambda bi: (bi, 0, 0)),
            pl.BlockSpec((HEADS, EMB_DIM, FEAT_DIM), lambda bi: (0, 0, 0)),
            pl.BlockSpec((1, FEAT_DIM), lambda bi: (0, 0)),
            pl.BlockSpec((1, HEADS, FEAT_DIM), lambda bi: (0, 0, 0)),
            pl.BlockSpec((1, HEADS, FEAT_DIM), lambda bi: (0, 0, 0)),
            pl.BlockSpec((EMB_DIM, HEADS * FEAT_DIM), lambda bi: (0, 0)),
            pl.BlockSpec((1, HEADS * FEAT_DIM), lambda bi: (0, 0)),
        ],
        out_specs=[
            pl.BlockSpec((1, HEADS, N, N), lambda bi: (bi, 0, 0, 0)),
            pl.BlockSpec((1, N, HEADS * FEAT_DIM), lambda bi: (bi, 0, 0)),
        ],
        out_shape=[
            jax.ShapeDtypeStruct((BATCH, HEADS, N, N), jnp.float32),
            jax.ShapeDtypeStruct((BATCH, N, HEADS * FEAT_DIM), jnp.float32),
        ],
        compiler_params=pltpu.CompilerParams(
            dimension_semantics=("parallel",),
        ),
    )(doc_sents_h, adj, W, b2, wsrc, wdst, Wh, bh2)
    return feat_out, attn
